# baseline (speedup 1.0000x reference)
import jax
import jax.numpy as jnp
from jax.experimental import pallas as pl
from jax.experimental.pallas import tpu as pltpu

_HOP = 320
_CONTRACT_LAST = (((1,), (1,)), ((), ()))
_CHUNK = 400


def _vq_body(frames_hbm, W1_hbm, b1_hbm, W2_hbm, b2_hbm, cb_hbm, out_ref,
             W1_v, b1_v, W2_v, b2_v, cb_v, f_v, cb2_v, sems):
    n_chunks = f_v.shape[0]
    c_W1 = pltpu.make_async_copy(W1_hbm, W1_v, sems.at[0])
    c_b1 = pltpu.make_async_copy(b1_hbm, b1_v, sems.at[1])
    c_W2 = pltpu.make_async_copy(W2_hbm, W2_v, sems.at[2])
    c_b2 = pltpu.make_async_copy(b2_hbm, b2_v, sems.at[3])
    c_cb = pltpu.make_async_copy(cb_hbm, cb_v, sems.at[4])
    c_f = [
        pltpu.make_async_copy(
            frames_hbm.at[pl.ds(j * _CHUNK, _CHUNK), :], f_v.at[j],
            sems.at[5 + j])
        for j in range(n_chunks)
    ]
    c_W1.start()
    c_b1.start()
    c_f[0].start()
    c_W2.start()
    c_b2.start()
    c_cb.start()
    for j in range(1, n_chunks):
        c_f[j].start()

    c_W1.wait()
    c_b1.wait()
    c_W2.wait()
    c_b2.wait()
    W1 = W1_v[...]
    b1 = b1_v[...]
    W2 = W2_v[...]
    b2 = b2_v[...]
    cb_ready = False
    for j in range(n_chunks):
        c_f[j].wait()
        f = f_v[j]
        h = jnp.tanh(
            jnp.dot(f, W1, preferred_element_type=jnp.float32) + b1)
        c = jnp.tanh(
            jnp.dot(h, W2, preferred_element_type=jnp.float32) + b2)
        z2 = jnp.sum(c * c, axis=1, keepdims=True)
        if not cb_ready:
            c_cb.wait()
            cb0 = cb_v[...]
            cb2_v[...] = jnp.sum(cb0 * cb0, axis=1, keepdims=True).T
            cb_ready = True
        cross = jax.lax.dot_general(c, cb_v[...], _CONTRACT_LAST,
                                    preferred_element_type=jnp.float32)
        s = z2 - 2.0 * cross + cb2_v[...]
        out_ref[0, pl.ds(j * _CHUNK, _CHUNK)] = jnp.argmin(s, axis=1).astype(
            jnp.int32)


def kernel(x, W1, b1, W2, b2, codebook):
    B = x.shape[0]
    if x.ndim == 3 and x.shape[-1] == 1:
        x = x[..., 0]
    T = x.shape[1] // _HOP
    M = B * T
    G, K, Dg = codebook.shape
    D = W2.shape[1]
    frames = x[:, : T * _HOP].reshape(M, _HOP)
    n_chunks = M // _CHUNK

    out = pl.pallas_call(
        _vq_body,
        in_specs=[
            pl.BlockSpec(memory_space=pl.ANY),
            pl.BlockSpec(memory_space=pl.ANY),
            pl.BlockSpec(memory_space=pl.ANY),
            pl.BlockSpec(memory_space=pl.ANY),
            pl.BlockSpec(memory_space=pl.ANY),
            pl.BlockSpec(memory_space=pl.ANY),
        ],
        out_shape=jax.ShapeDtypeStruct((1, M), jnp.int32),
        scratch_shapes=[
            pltpu.VMEM((_HOP, D), jnp.float32),
            pltpu.VMEM((1, D), jnp.float32),
            pltpu.VMEM((D, D), jnp.float32),
            pltpu.VMEM((1, D), jnp.float32),
            pltpu.VMEM((K, Dg), jnp.float32),
            pltpu.VMEM((n_chunks, _CHUNK, _HOP), jnp.float32),
            pltpu.VMEM((1, K), jnp.float32),
            pltpu.SemaphoreType.DMA((5 + n_chunks,)),
        ],
    )(frames, W1, b1[None], W2, b2[None], codebook[0])
    return out.reshape(B, T, G).astype(jnp.int32)


# speedup vs baseline: 1.0106x; 1.0106x over previous
import jax
import jax.numpy as jnp
from jax.experimental import pallas as pl
from jax.experimental.pallas import tpu as pltpu

_HOP = 320
_CONTRACT_LAST = (((1,), (1,)), ((), ()))
_CHUNK = 400


def _vq_body(frames_hbm, W1_hbm, b1_hbm, W2_hbm, b2_hbm, cb_hbm, out_ref,
             W1_v, b1_v, W2_v, b2_v, cb_v, f_v, cb2_v, sems):
    n_chunks = f_v.shape[0]
    c_W1 = pltpu.make_async_copy(W1_hbm, W1_v, sems.at[0])
    c_b1 = pltpu.make_async_copy(b1_hbm, b1_v, sems.at[1])
    c_W2 = pltpu.make_async_copy(W2_hbm, W2_v, sems.at[2])
    c_b2 = pltpu.make_async_copy(b2_hbm, b2_v, sems.at[3])
    c_cb = pltpu.make_async_copy(cb_hbm, cb_v, sems.at[4])
    c_f = [
        pltpu.make_async_copy(
            frames_hbm.at[pl.ds(j * _CHUNK, _CHUNK), :], f_v.at[j],
            sems.at[5 + j])
        for j in range(n_chunks)
    ]
    c_W1.start()
    c_b1.start()
    c_f[0].start()
    c_W2.start()
    c_b2.start()
    c_cb.start()
    for j in range(1, n_chunks):
        c_f[j].start()

    c_W1.wait()
    c_b1.wait()
    for j in range(n_chunks):
        c_f[j].wait()
        f = f_v[j]
        h = jnp.tanh(
            jnp.dot(f, W1_v[...], preferred_element_type=jnp.float32)
            + b1_v[...])
        if j == 0:
            c_W2.wait()
            c_b2.wait()
        c = jnp.tanh(
            jnp.dot(h, W2_v[...], preferred_element_type=jnp.float32)
            + b2_v[...])
        z2 = jnp.sum(c * c, axis=1, keepdims=True)
        if j == 0:
            c_cb.wait()
            cb0 = cb_v[...]
            cb2_v[...] = jnp.sum(cb0 * cb0, axis=1, keepdims=True).T
        cross = jax.lax.dot_general(c, cb_v[...], _CONTRACT_LAST,
                                    preferred_element_type=jnp.float32)
        s = z2 - 2.0 * cross + cb2_v[...]
        out_ref[0, pl.ds(j * _CHUNK, _CHUNK)] = jnp.argmin(s, axis=1).astype(
            jnp.int32)


def kernel(x, W1, b1, W2, b2, codebook):
    B = x.shape[0]
    if x.ndim == 3 and x.shape[-1] == 1:
        x = x[..., 0]
    T = x.shape[1] // _HOP
    M = B * T
    G, K, Dg = codebook.shape
    D = W2.shape[1]
    frames = x[:, : T * _HOP].reshape(M, _HOP)
    n_chunks = M // _CHUNK

    out = pl.pallas_call(
        _vq_body,
        in_specs=[
            pl.BlockSpec(memory_space=pl.ANY),
            pl.BlockSpec(memory_space=pl.ANY),
            pl.BlockSpec(memory_space=pl.ANY),
            pl.BlockSpec(memory_space=pl.ANY),
            pl.BlockSpec(memory_space=pl.ANY),
            pl.BlockSpec(memory_space=pl.ANY),
        ],
        out_shape=jax.ShapeDtypeStruct((1, M), jnp.int32),
        scratch_shapes=[
            pltpu.VMEM((_HOP, D), jnp.float32),
            pltpu.VMEM((1, D), jnp.float32),
            pltpu.VMEM((D, D), jnp.float32),
            pltpu.VMEM((1, D), jnp.float32),
            pltpu.VMEM((K, Dg), jnp.float32),
            pltpu.VMEM((n_chunks, _CHUNK, _HOP), jnp.float32),
            pltpu.VMEM((1, K), jnp.float32),
            pltpu.SemaphoreType.DMA((5 + n_chunks,)),
        ],
    )(frames, W1, b1[None], W2, b2[None], codebook[0])
    return out.reshape(B, T, G).astype(jnp.int32)


# speedup vs baseline: 1.0833x; 1.0719x over previous
import jax
import jax.numpy as jnp
from jax.experimental import pallas as pl
from jax.experimental.pallas import tpu as pltpu

_HOP = 320
_CONTRACT_LAST = (((1,), (1,)), ((), ()))
_CHUNK = 400
_LANES = 128


def _argmin_rows(s):
    rows, k = s.shape
    n_l = k // _LANES
    chunks = [s[:, a * _LANES:(a + 1) * _LANES] for a in range(n_l)]
    v = chunks[0]
    for a in range(1, n_l):
        v = jnp.minimum(v, chunks[a])
    m = jnp.min(v, axis=1, keepdims=True)
    lane = jax.lax.broadcasted_iota(jnp.int32, (rows, _LANES), 1)
    big = jnp.full((rows, _LANES), k, dtype=jnp.int32)
    idx = big
    for a in range(n_l):
        idx = jnp.minimum(idx, jnp.where(chunks[a] == m, a * _LANES + lane,
                                         big))
    return jnp.min(idx, axis=1)


def _vq_body(frames_ref, W1_ref, b1_ref, W2_ref, b2_ref, cb_ref, out_ref,
             cb2_ref):
    cb0 = cb_ref[...]
    cb2_ref[...] = jnp.sum(cb0 * cb0, axis=1, keepdims=True).T

    W1 = W1_ref[...]
    W2 = W2_ref[...]
    b1 = b1_ref[...]
    b2 = b2_ref[...]
    cb = cb_ref[...]
    cb2 = cb2_ref[...]
    mt = frames_ref.shape[0]
    for j in range(0, mt, _CHUNK):
        f = frames_ref[pl.ds(j, _CHUNK), :]
        h = jnp.tanh(
            jnp.dot(f, W1, preferred_element_type=jnp.float32) + b1)
        c = jnp.tanh(
            jnp.dot(h, W2, preferred_element_type=jnp.float32) + b2)
        z2 = jnp.sum(c * c, axis=1, keepdims=True)
        cross = jax.lax.dot_general(c, cb, _CONTRACT_LAST,
                                    preferred_element_type=jnp.float32)
        s = z2 - 2.0 * cross + cb2
        out_ref[0, 0, pl.ds(j, _CHUNK)] = jnp.argmin(s, axis=1).astype(
            jnp.int32)


def kernel(x, W1, b1, W2, b2, codebook):
    B = x.shape[0]
    if x.ndim == 3 and x.shape[-1] == 1:
        x = x[..., 0]
    T = x.shape[1] // _HOP
    M = B * T
    G, K, Dg = codebook.shape
    D = W2.shape[1]
    frames = x[:, : T * _HOP].reshape(M, _HOP)

    MT = M
    grid = M // MT
    out = pl.pallas_call(
        _vq_body,
        grid=(grid,),
        in_specs=[
            pl.BlockSpec((MT, _HOP), lambda i: (i, 0)),
            pl.BlockSpec((_HOP, D), lambda i: (0, 0)),
            pl.BlockSpec((1, D), lambda i: (0, 0)),
            pl.BlockSpec((D, D), lambda i: (0, 0)),
            pl.BlockSpec((1, D), lambda i: (0, 0)),
            pl.BlockSpec((K, Dg), lambda i: (0, 0)),
        ],
        out_specs=pl.BlockSpec((1, 1, MT), lambda i: (i, 0, 0)),
        out_shape=jax.ShapeDtypeStruct((grid, 1, MT), jnp.int32),
        scratch_shapes=[pltpu.VMEM((1, K), jnp.float32)],
    )(frames, W1, b1[None], W2, b2[None], codebook[0])
    return out.reshape(B, T, G).astype(jnp.int32)
